# baseline (device time: 88706 ns/iter reference)
import jax
import jax.numpy as jnp
from jax import lax
from jax.experimental import pallas as pl
from jax.experimental.pallas import tpu as pltpu


def kernel(x, assign, W1, W2):
    T, D = x.shape
    E_loc, _, F = W1.shape
    E_pair = 2
    K = 576
    C = 192
    NC = K // C

    my_p = lax.axis_index("x")
    my_q = lax.axis_index("y")

    xb = x.astype(jnp.bfloat16)

    pair = assign // E_pair
    pp, qq = pair // 2, pair % 2
    g = 2 * (qq != my_q).astype(jnp.int32) + (pp != my_p).astype(jnp.int32)

    onehot = (g[:, None] == jnp.arange(4)[None, :]).astype(jnp.int32)
    rank = jnp.sum((jnp.cumsum(onehot, axis=0) - onehot) * onehot, axis=1)
    rank = jnp.minimum(rank, K - 1)
    pos_col = (g * K + rank).astype(jnp.int32).reshape(T, 1)
    rank0_row = jnp.where(g == 0, rank, -1).astype(jnp.int32).reshape(1, T)
    rank1_row = jnp.where(g == 1, rank, -1).astype(jnp.int32).reshape(1, T)
    a_col = assign.astype(jnp.bfloat16).reshape(T, 1)

    def body(
        x_ref, a_ref, r0_ref, r1_ref, pos_ref, w1_any, w2_any, out_ref,
        parts_ref, xs_ref, as_ref, xr_ref, ar_ref, po_ref,
        w1_ref, w2_ref, w1s_ref, w2s_ref, wsem,
        s1send, s1recv, psend, precv, fsend, frecv, dsend, drecv,
    ):
        my_x = lax.axis_index("x")
        my_y = lax.axis_index("y")
        x_peer = (1 - my_x, my_y)
        y_peer = (my_x, 1 - my_y)
        diag = (1 - my_x, 1 - my_y)

        W_STEPS = [(e, h) for e in range(E_pair) for h in range(4)]

        def w1_copy(i):
            e, h = W_STEPS[i]
            return pltpu.make_async_copy(
                w1_any.at[my_y * E_pair + e, pl.ds(h * (D // 4), D // 4)],
                w1s_ref.at[i % 2], wsem.at[0, i % 2],
            )

        def w2_copy(i):
            e, h = W_STEPS[i]
            return pltpu.make_async_copy(
                w2_any.at[my_y * E_pair + e, pl.ds(h * (F // 4), F // 4)],
                w2s_ref.at[i % 2], wsem.at[1, i % 2],
            )

        for i in range(2):
            w1_copy(i).start()
            w2_copy(i).start()

        def w_drain():
            for i in range(len(W_STEPS)):
                e, h = W_STEPS[i]
                w1_copy(i).wait()
                w1_ref[e, pl.ds(h * (D // 4), D // 4), :] = w1s_ref[
                    i % 2].astype(jnp.bfloat16)
                w2_copy(i).wait()
                w2_ref[e, pl.ds(h * (F // 4), F // 4), :] = w2s_ref[
                    i % 2].astype(jnp.bfloat16)
                if i + 2 < len(W_STEPS):
                    w1_copy(i + 2).start()
                    w2_copy(i + 2).start()

        barrier_sem = pltpu.get_barrier_semaphore()
        for nbr in (x_peer, y_peer):
            pl.semaphore_signal(
                barrier_sem, inc=1,
                device_id=nbr, device_id_type=pl.DeviceIdType.MESH,
            )

        def sel_matrix(rank_row):
            s_iota = lax.broadcasted_iota(jnp.int32, (K, T), 0)
            return (s_iota == rank_row).astype(jnp.bfloat16)

        def dispatch(S, v_ref):
            return jnp.dot(
                S, v_ref[...], preferred_element_type=jnp.float32
            ).astype(jnp.bfloat16)

        S1 = sel_matrix(r1_ref[...])
        xs_ref[...] = dispatch(S1, x_ref)
        as_ref[...] = dispatch(S1, a_ref)

        pl.semaphore_wait(barrier_sem, 2)

        r_x = pltpu.make_async_remote_copy(
            src_ref=xs_ref, dst_ref=xr_ref,
            send_sem=s1send.at[0], recv_sem=s1recv.at[0],
            device_id=x_peer, device_id_type=pl.DeviceIdType.MESH,
        )
        r_a = pltpu.make_async_remote_copy(
            src_ref=as_ref, dst_ref=ar_ref,
            send_sem=s1send.at[1], recv_sem=s1recv.at[1],
            device_id=x_peer, device_id_type=pl.DeviceIdType.MESH,
        )
        r_x.start()
        r_a.start()

        def part_chunk(tok, asg):
            acc = jnp.zeros((C, D), jnp.float32)
            for e in range(E_pair):
                ge = (my_x * E_loc + my_y * E_pair + e).astype(jnp.float32)
                h = jnp.maximum(
                    jnp.dot(tok, w1_ref[e], preferred_element_type=jnp.float32),
                    0.0,
                ).astype(jnp.bfloat16)
                y = jnp.dot(h, w2_ref[e], preferred_element_type=jnp.float32)
                acc = acc + jnp.where(asg.astype(jnp.float32) == ge, y, 0.0)
            return acc

        S0 = sel_matrix(r0_ref[...])
        xm = dispatch(S0, x_ref)
        am = dispatch(S0, a_ref)
        w_drain()
        for c in range(NC):
            sl = pl.ds(c * C, C)
            parts_ref[sl, :] = part_chunk(xm[c * C:(c + 1) * C, :],
                                          am[c * C:(c + 1) * C, :]).astype(
                jnp.bfloat16
            )

        r_yo = pltpu.make_async_remote_copy(
            src_ref=parts_ref.at[pl.ds(0, K)],
            dst_ref=parts_ref.at[pl.ds(2 * K, K)],
            send_sem=fsend.at[0], recv_sem=frecv.at[0],
            device_id=y_peer, device_id_type=pl.DeviceIdType.MESH,
        )
        r_yo.start()

        def combine_cols(col0, ncols, accumulate):
            for r in range(4):
                rsl = pl.ds(r * (T // 4), T // 4)
                p_iota = (
                    lax.broadcasted_iota(jnp.int32, (T // 4, ncols), 1) + col0
                )
                Pb = (p_iota == pos_ref[rsl, :]).astype(jnp.bfloat16)
                contrib = jnp.dot(
                    Pb, parts_ref[pl.ds(col0, ncols), :],
                    preferred_element_type=jnp.float32,
                )
                if accumulate:
                    out_ref[rsl, :] = out_ref[rsl, :] + contrib
                else:
                    out_ref[rsl, :] = contrib

        def combine(b, accumulate):
            combine_cols(b * K, K, accumulate)

        combine(0, accumulate=False)

        r_x.wait()
        r_a.wait()

        rps = []
        for c in range(NC):
            sl = pl.ds(c * C, C)
            po_ref[sl, :] = part_chunk(xr_ref[sl, :], ar_ref[sl, :]).astype(
                jnp.bfloat16
            )
            rp = pltpu.make_async_remote_copy(
                src_ref=po_ref.at[sl],
                dst_ref=parts_ref.at[pl.ds(K + c * C, C)],
                send_sem=psend.at[c], recv_sem=precv.at[c],
                device_id=x_peer, device_id_type=pl.DeviceIdType.MESH,
            )
            rp.start()
            rps.append(rp)

        r_yo.wait()
        combine(2, accumulate=True)

        rfs = []
        for c in range(NC):
            rps[c].wait()
            rf = pltpu.make_async_remote_copy(
                src_ref=parts_ref.at[pl.ds(K + c * C, C)],
                dst_ref=parts_ref.at[pl.ds(3 * K + c * C, C)],
                send_sem=dsend.at[c], recv_sem=drecv.at[c],
                device_id=y_peer, device_id_type=pl.DeviceIdType.MESH,
            )
            rf.start()
            rfs.append(rf)

        combine(1, accumulate=True)

        for rf in rfs:
            rf.wait()
        combine(3, accumulate=True)

    return pl.pallas_call(
        body,
        out_shape=jax.ShapeDtypeStruct((T, D), jnp.float32),
        in_specs=[pl.BlockSpec(memory_space=pltpu.VMEM)] * 5
        + [pl.BlockSpec(memory_space=pl.ANY)] * 2,
        out_specs=pl.BlockSpec(memory_space=pltpu.VMEM),
        scratch_shapes=[
            pltpu.VMEM((4 * K, D), jnp.bfloat16),
            pltpu.VMEM((K, D), jnp.bfloat16),
            pltpu.VMEM((K, 1), jnp.bfloat16),
            pltpu.VMEM((K, D), jnp.bfloat16),
            pltpu.VMEM((K, 1), jnp.bfloat16),
            pltpu.VMEM((K, D), jnp.bfloat16),
            pltpu.VMEM((E_pair, D, F), jnp.bfloat16),
            pltpu.VMEM((E_pair, F, D), jnp.bfloat16),
            pltpu.VMEM((2, D // 4, F), jnp.float32),
            pltpu.VMEM((2, F // 4, D), jnp.float32),
            pltpu.SemaphoreType.DMA((2, 2)),
            pltpu.SemaphoreType.DMA((2,)),
            pltpu.SemaphoreType.DMA((2,)),
            pltpu.SemaphoreType.DMA((NC,)),
            pltpu.SemaphoreType.DMA((NC,)),
            pltpu.SemaphoreType.DMA((1,)),
            pltpu.SemaphoreType.DMA((1,)),
            pltpu.SemaphoreType.DMA((NC,)),
            pltpu.SemaphoreType.DMA((NC,)),
        ],
        compiler_params=pltpu.CompilerParams(
            collective_id=0, vmem_limit_bytes=63 * 1024 * 1024
        ),
    )(xb, a_col, rank0_row, rank1_row, pos_col, W1, W2)


# device time: 88041 ns/iter; 1.0076x vs baseline; 1.0076x over previous
import jax
import jax.numpy as jnp
from jax import lax
from jax.experimental import pallas as pl
from jax.experimental.pallas import tpu as pltpu


def kernel(x, assign, W1, W2):
    T, D = x.shape
    E_loc, _, F = W1.shape
    E_pair = 2
    K = 576
    C = 288
    NC = K // C

    my_p = lax.axis_index("x")
    my_q = lax.axis_index("y")

    xb = x.astype(jnp.bfloat16)

    pair = assign // E_pair
    pp, qq = pair // 2, pair % 2
    g = 2 * (qq != my_q).astype(jnp.int32) + (pp != my_p).astype(jnp.int32)

    onehot = (g[:, None] == jnp.arange(4)[None, :]).astype(jnp.int32)
    rank = jnp.sum((jnp.cumsum(onehot, axis=0) - onehot) * onehot, axis=1)
    rank = jnp.minimum(rank, K - 1)
    pos_col = (g * K + rank).astype(jnp.int32).reshape(T, 1)
    rank0_row = jnp.where(g == 0, rank, -1).astype(jnp.int32).reshape(1, T)
    rank1_row = jnp.where(g == 1, rank, -1).astype(jnp.int32).reshape(1, T)
    a_col = assign.astype(jnp.bfloat16).reshape(T, 1)

    def body(
        x_ref, a_ref, r0_ref, r1_ref, pos_ref, w1_any, w2_any, out_ref,
        parts_ref, xs_ref, as_ref, xr_ref, ar_ref, po_ref,
        w1_ref, w2_ref, w1s_ref, w2s_ref, wsem,
        s1send, s1recv, psend, precv, fsend, frecv, dsend, drecv,
    ):
        my_x = lax.axis_index("x")
        my_y = lax.axis_index("y")
        x_peer = (1 - my_x, my_y)
        y_peer = (my_x, 1 - my_y)
        diag = (1 - my_x, 1 - my_y)

        W_STEPS = [(e, h) for e in range(E_pair) for h in range(4)]

        def w1_copy(i):
            e, h = W_STEPS[i]
            return pltpu.make_async_copy(
                w1_any.at[my_y * E_pair + e, pl.ds(h * (D // 4), D // 4)],
                w1s_ref.at[i % 2], wsem.at[0, i % 2],
            )

        def w2_copy(i):
            e, h = W_STEPS[i]
            return pltpu.make_async_copy(
                w2_any.at[my_y * E_pair + e, pl.ds(h * (F // 4), F // 4)],
                w2s_ref.at[i % 2], wsem.at[1, i % 2],
            )

        for i in range(2):
            w1_copy(i).start()
            w2_copy(i).start()

        def w_drain():
            for i in range(len(W_STEPS)):
                e, h = W_STEPS[i]
                w1_copy(i).wait()
                w1_ref[e, pl.ds(h * (D // 4), D // 4), :] = w1s_ref[
                    i % 2].astype(jnp.bfloat16)
                w2_copy(i).wait()
                w2_ref[e, pl.ds(h * (F // 4), F // 4), :] = w2s_ref[
                    i % 2].astype(jnp.bfloat16)
                if i + 2 < len(W_STEPS):
                    w1_copy(i + 2).start()
                    w2_copy(i + 2).start()

        barrier_sem = pltpu.get_barrier_semaphore()
        for nbr in (x_peer, y_peer):
            pl.semaphore_signal(
                barrier_sem, inc=1,
                device_id=nbr, device_id_type=pl.DeviceIdType.MESH,
            )

        def sel_matrix(rank_row):
            s_iota = lax.broadcasted_iota(jnp.int32, (K, T), 0)
            return (s_iota == rank_row).astype(jnp.bfloat16)

        def dispatch(S, v_ref):
            return jnp.dot(
                S, v_ref[...], preferred_element_type=jnp.float32
            ).astype(jnp.bfloat16)

        S1 = sel_matrix(r1_ref[...])
        xs_ref[...] = dispatch(S1, x_ref)
        as_ref[...] = dispatch(S1, a_ref)

        pl.semaphore_wait(barrier_sem, 2)

        r_x = pltpu.make_async_remote_copy(
            src_ref=xs_ref, dst_ref=xr_ref,
            send_sem=s1send.at[0], recv_sem=s1recv.at[0],
            device_id=x_peer, device_id_type=pl.DeviceIdType.MESH,
        )
        r_a = pltpu.make_async_remote_copy(
            src_ref=as_ref, dst_ref=ar_ref,
            send_sem=s1send.at[1], recv_sem=s1recv.at[1],
            device_id=x_peer, device_id_type=pl.DeviceIdType.MESH,
        )
        r_x.start()
        r_a.start()

        def part_chunk(tok, asg):
            acc = jnp.zeros((C, D), jnp.float32)
            for e in range(E_pair):
                ge = (my_x * E_loc + my_y * E_pair + e).astype(jnp.float32)
                h = jnp.maximum(
                    jnp.dot(tok, w1_ref[e], preferred_element_type=jnp.float32),
                    0.0,
                ).astype(jnp.bfloat16)
                y = jnp.dot(h, w2_ref[e], preferred_element_type=jnp.float32)
                acc = acc + jnp.where(asg.astype(jnp.float32) == ge, y, 0.0)
            return acc

        S0 = sel_matrix(r0_ref[...])
        xm = dispatch(S0, x_ref)
        am = dispatch(S0, a_ref)
        w_drain()
        for c in range(NC):
            sl = pl.ds(c * C, C)
            parts_ref[sl, :] = part_chunk(xm[c * C:(c + 1) * C, :],
                                          am[c * C:(c + 1) * C, :]).astype(
                jnp.bfloat16
            )

        r_yo = pltpu.make_async_remote_copy(
            src_ref=parts_ref.at[pl.ds(0, K)],
            dst_ref=parts_ref.at[pl.ds(2 * K, K)],
            send_sem=fsend.at[0], recv_sem=frecv.at[0],
            device_id=y_peer, device_id_type=pl.DeviceIdType.MESH,
        )
        r_yo.start()

        def combine_cols(col0, ncols, accumulate):
            for r in range(4):
                rsl = pl.ds(r * (T // 4), T // 4)
                p_iota = (
                    lax.broadcasted_iota(jnp.int32, (T // 4, ncols), 1) + col0
                )
                Pb = (p_iota == pos_ref[rsl, :]).astype(jnp.bfloat16)
                contrib = jnp.dot(
                    Pb, parts_ref[pl.ds(col0, ncols), :],
                    preferred_element_type=jnp.float32,
                )
                if accumulate:
                    out_ref[rsl, :] = out_ref[rsl, :] + contrib
                else:
                    out_ref[rsl, :] = contrib

        def combine(b, accumulate):
            combine_cols(b * K, K, accumulate)

        combine(0, accumulate=False)

        r_x.wait()
        r_a.wait()

        rps = []
        for c in range(NC):
            sl = pl.ds(c * C, C)
            po_ref[sl, :] = part_chunk(xr_ref[sl, :], ar_ref[sl, :]).astype(
                jnp.bfloat16
            )
            rp = pltpu.make_async_remote_copy(
                src_ref=po_ref.at[sl],
                dst_ref=parts_ref.at[pl.ds(K + c * C, C)],
                send_sem=psend.at[c], recv_sem=precv.at[c],
                device_id=x_peer, device_id_type=pl.DeviceIdType.MESH,
            )
            rp.start()
            rps.append(rp)

        r_yo.wait()
        combine(2, accumulate=True)

        rfs = []
        for c in range(NC):
            rps[c].wait()
            rf = pltpu.make_async_remote_copy(
                src_ref=parts_ref.at[pl.ds(K + c * C, C)],
                dst_ref=parts_ref.at[pl.ds(3 * K + c * C, C)],
                send_sem=dsend.at[c], recv_sem=drecv.at[c],
                device_id=y_peer, device_id_type=pl.DeviceIdType.MESH,
            )
            rf.start()
            rfs.append(rf)

        combine(1, accumulate=True)

        for rf in rfs:
            rf.wait()
        combine(3, accumulate=True)

    return pl.pallas_call(
        body,
        out_shape=jax.ShapeDtypeStruct((T, D), jnp.float32),
        in_specs=[pl.BlockSpec(memory_space=pltpu.VMEM)] * 5
        + [pl.BlockSpec(memory_space=pl.ANY)] * 2,
        out_specs=pl.BlockSpec(memory_space=pltpu.VMEM),
        scratch_shapes=[
            pltpu.VMEM((4 * K, D), jnp.bfloat16),
            pltpu.VMEM((K, D), jnp.bfloat16),
            pltpu.VMEM((K, 1), jnp.bfloat16),
            pltpu.VMEM((K, D), jnp.bfloat16),
            pltpu.VMEM((K, 1), jnp.bfloat16),
            pltpu.VMEM((K, D), jnp.bfloat16),
            pltpu.VMEM((E_pair, D, F), jnp.bfloat16),
            pltpu.VMEM((E_pair, F, D), jnp.bfloat16),
            pltpu.VMEM((2, D // 4, F), jnp.float32),
            pltpu.VMEM((2, F // 4, D), jnp.float32),
            pltpu.SemaphoreType.DMA((2, 2)),
            pltpu.SemaphoreType.DMA((2,)),
            pltpu.SemaphoreType.DMA((2,)),
            pltpu.SemaphoreType.DMA((NC,)),
            pltpu.SemaphoreType.DMA((NC,)),
            pltpu.SemaphoreType.DMA((1,)),
            pltpu.SemaphoreType.DMA((1,)),
            pltpu.SemaphoreType.DMA((NC,)),
            pltpu.SemaphoreType.DMA((NC,)),
        ],
        compiler_params=pltpu.CompilerParams(
            collective_id=0, vmem_limit_bytes=63 * 1024 * 1024
        ),
    )(xb, a_col, rank0_row, rank1_row, pos_col, W1, W2)
